# baseline (speedup 1.0000x reference)
import functools

import jax
import jax.numpy as jnp
from jax import lax
from jax.experimental import pallas as pl
from jax.experimental.pallas import tpu as pltpu
from jax.experimental.pallas import tpu_sc as plsc

B = 16384
H = 200
E = 8
D = 4
N = B * H
NW = 32
HB = H // 8
BB = B // 128
BB_W = BB // NW

_mesh = plsc.VectorSubcoreMesh(core_axis_name="c", subcore_axis_name="s")


@functools.partial(
    pl.kernel,
    out_type=jax.ShapeDtypeStruct((H * BB * D, 128), jnp.float32),
    mesh=_mesh,
    compiler_params=pltpu.CompilerParams(
        needs_layout_passes=False,
        disable_bounds_checks=True,
        disable_semaphore_checks=True,
        skip_device_barrier=True,
    ),
    scratch_types=[
        pltpu.VMEM((E * D,), jnp.float32),
        pltpu.VMEM((32, 128), jnp.int32),
        pltpu.VMEM((32, 128), jnp.int32),
        pltpu.VMEM((8, 16, 128), jnp.float32),
        pltpu.VMEM((8, 16, 128), jnp.float32),
        pltpu.SemaphoreType.DMA,
        pltpu.SemaphoreType.DMA,
        pltpu.SemaphoreType.DMA,
        pltpu.SemaphoreType.DMA,
        pltpu.SemaphoreType.DMA,
    ],
)
def _embed_sc(idx_hbm, tab_hbm, out_hbm, tab_v, idx_v0, idx_v1,
              out_v0, out_v1, tsem, isem0, isem1, osem0, osem1):
    wid = lax.axis_index("s") * 2 + lax.axis_index("c")

    pltpu.async_copy(tab_hbm, tab_v, tsem).wait()

    idx_bufs = (idx_v0, idx_v1)
    out_bufs = (out_v0, out_v1)
    isems = (isem0, isem1)
    osems = (osem0, osem1)

    icopies = [None, None]
    ocopies = [[], []]
    icopies[0] = pltpu.async_copy(
        idx_hbm.at[pl.ds(wid * 32, 32), :], idx_bufs[0], isems[0])

    for s in range(HB):
        bf = s % 2
        nb = (s + 1) % 2
        if s + 1 < HB:
            icopies[nb] = pltpu.async_copy(
                idx_hbm.at[pl.ds((s + 1) * 1024 + wid * 32, 32), :],
                idx_bufs[nb], isems[nb])
        icopies[bf].wait()
        for c in ocopies[bf]:
            c.wait()
        ocopies[bf] = []
        ib = idx_bufs[bf]
        ob = out_bufs[bf]

        @plsc.parallel_loop(0, 256, unroll=8)
        def _(g, ib=ib, ob=ob):
            r = g >> 3
            blg = (g & 7) * 16
            bbl = r >> 3
            hl = r & 7
            i16 = ib[r, pl.ds(blg, 16)]
            i4 = i16 * D
            for d in range(D):
                vals = plsc.load_gather(tab_v, [i4 + d])
                ob[hl, bbl * D + d, pl.ds(blg, 16)] = vals

        for hl in range(8):
            ocopies[bf].append(pltpu.async_copy(
                ob.at[hl],
                out_hbm.at[pl.ds((8 * s + hl) * 512 + wid * 16, 16), :],
                osems[bf]))

    for cs in ocopies:
        for c in cs:
            c.wait()


def kernel(indices, table):
    idx_lin = (indices.reshape(BB, 128, HB, 8)
               .transpose(2, 0, 3, 1)
               .reshape(HB * BB * 8, 128))
    out_lin = _embed_sc(idx_lin, table.reshape(E * D))
    return (out_lin.reshape(H, BB, D, 128)
            .transpose(1, 3, 0, 2)
            .reshape(B, H, D))


# speedup vs baseline: 1.0535x; 1.0535x over previous
import functools

import jax
import jax.numpy as jnp
from jax import lax
from jax.experimental import pallas as pl
from jax.experimental.pallas import tpu as pltpu
from jax.experimental.pallas import tpu_sc as plsc

B = 16384
H = 200
E = 8
D = 4
NW = 32
HB = H // 8
BB = B // 128

_mesh = plsc.VectorSubcoreMesh(core_axis_name="c", subcore_axis_name="s")


@functools.partial(
    pl.kernel,
    out_type=jax.ShapeDtypeStruct((H, NW, 16, 128), jnp.float32),
    mesh=_mesh,
    compiler_params=pltpu.CompilerParams(needs_layout_passes=False),
    scratch_types=[
        pltpu.VMEM((E * D,), jnp.float32),
        pltpu.VMEM((32, 128), jnp.int32),
        pltpu.VMEM((32, 128), jnp.int32),
        pltpu.VMEM((8, 16, 128), jnp.float32),
        pltpu.VMEM((8, 16, 128), jnp.float32),
        pltpu.SemaphoreType.DMA,
        pltpu.SemaphoreType.DMA,
        pltpu.SemaphoreType.DMA,
        pltpu.SemaphoreType.DMA,
        pltpu.SemaphoreType.DMA,
    ],
)
def _embed_sc(idx_hbm, tab_hbm, out_hbm, tab_v, idx_v0, idx_v1,
              out_v0, out_v1, tsem, isem0, isem1, osem0, osem1):
    wid = lax.axis_index("s") * 2 + lax.axis_index("c")

    pltpu.async_copy(tab_hbm, tab_v, tsem).wait()

    idx_bufs = (idx_v0, idx_v1)
    out_bufs = (out_v0, out_v1)
    isems = (isem0, isem1)
    osems = (osem0, osem1)

    icopies = [None, None]
    ocopies = [None, None]
    icopies[0] = pltpu.async_copy(
        idx_hbm.at[0, wid], idx_bufs[0], isems[0])

    for s in range(HB):
        bf = s % 2
        nb = (s + 1) % 2
        if s + 1 < HB:
            icopies[nb] = pltpu.async_copy(
                idx_hbm.at[s + 1, wid], idx_bufs[nb], isems[nb])
        icopies[bf].wait()
        if ocopies[bf] is not None:
            ocopies[bf].wait()
        ib = idx_bufs[bf]
        ob = out_bufs[bf]

        @plsc.parallel_loop(0, 256, unroll=8)
        def _(g, ib=ib, ob=ob):
            r = g >> 3
            blg = (g & 7) * 16
            bbl = r >> 3
            hl = r & 7
            i16 = ib[r, pl.ds(blg, 16)]
            i4 = i16 * D
            for d in range(D):
                vals = plsc.load_gather(tab_v, [i4 + d])
                ob[hl, bbl * D + d, pl.ds(blg, 16)] = vals

        ocopies[bf] = pltpu.async_copy(
            ob, out_hbm.at[pl.ds(8 * s, 8), wid], osems[bf])

    for c in ocopies:
        if c is not None:
            c.wait()


def kernel(indices, table):
    idx_lin = (indices.reshape(BB, 128, HB, 8)
               .transpose(2, 0, 3, 1)
               .reshape(HB, NW, 32, 128))
    out_lin = _embed_sc(idx_lin, table.reshape(E * D))
    return (out_lin.reshape(H, BB, D, 128)
            .transpose(1, 3, 0, 2)
            .reshape(B, H, D))
